# baseline (speedup 1.0000x reference)
import jax
import jax.numpy as jnp
from jax.experimental import pallas as pl
from jax.experimental.pallas import tpu as pltpu

_CHUNK = 2048


def _occ_body(bi_ref, ri_ref, occ_ref):
    L = bi_ref.shape[0]
    occ_ref[...] = jnp.zeros_like(occ_ref)
    lane = jax.lax.broadcasted_iota(jnp.int32, (1, 256), 1)

    def body(c, carry):
        bc = bi_ref[pl.ds(c * _CHUNK, _CHUNK), :]
        rc = ri_ref[pl.ds(c * _CHUNK, _CHUNK), :]
        ids = bc * 16 + rc
        hit = (ids == lane).astype(jnp.float32)
        occ_ref[0:1, :] = jnp.maximum(
            occ_ref[0:1, :], jnp.max(hit, axis=0, keepdims=True))
        return carry

    jax.lax.fori_loop(0, L // _CHUNK, body, 0)


def _dense_body(x_ref, occ_ref, roots_ref, w_ref, u_ref, out_ref, g_ref):
    b = pl.program_id(0)
    H = w_ref.shape[0]
    wb = jnp.broadcast_to(w_ref[...], (H, H))

    x_blk = x_ref[0, 16:, :]
    gpre = jnp.dot(x_blk, wb, preferred_element_type=jnp.float32)
    g = jax.nn.sigmoid(gpre)
    out_ref[0, 16:, :] = x_blk * g
    g_ref[0, 16:, :] = g[:, 0:1]

    sub = jax.lax.broadcasted_iota(jnp.int32, (16, 256), 0)
    lane = jax.lax.broadcasted_iota(jnp.int32, (16, 256), 1)
    sel = (lane == sub + b * 16).astype(jnp.float32)
    m16 = jnp.sum(sel * occ_ref[0:1, :], axis=1, keepdims=True)
    ohb = (jax.lax.broadcasted_iota(jnp.int32, (1, 16), 1) == b
           ).astype(jnp.float32)
    rv = jnp.dot(ohb, roots_ref[...], preferred_element_type=jnp.float32)
    a_b = jnp.sum(jnp.dot(rv, u_ref[...], preferred_element_type=jnp.float32))
    x16 = x_ref[0, 0:16, :]
    gpre16 = jnp.dot(x16, wb, preferred_element_type=jnp.float32)
    g16 = jax.nn.sigmoid(gpre16 + m16 * a_b)
    out_ref[0, 0:16, :] = x16 * g16 + (m16 * (1.0 - g16)) * rv
    g_ref[0, 0:16, :] = g16[:, 0:1]


def kernel(x, mask_nonzero, w, u):
    B, N, H = x.shape
    L = mask_nonzero.shape[1]
    bi = mask_nonzero[0].reshape(L, 1)
    ri = mask_nonzero[1].reshape(L, 1)
    roots = x[:, 0, :]

    occ = pl.pallas_call(
        _occ_body,
        out_shape=jax.ShapeDtypeStruct((8, 256), jnp.float32),
    )(bi, ri)

    out, g = pl.pallas_call(
        _dense_body,
        grid=(B,),
        in_specs=[
            pl.BlockSpec((1, N, H), lambda b: (b, 0, 0)),
            pl.BlockSpec((8, 256), lambda b: (0, 0)),
            pl.BlockSpec((B, H), lambda b: (0, 0)),
            pl.BlockSpec((H, 1), lambda b: (0, 0)),
            pl.BlockSpec((H, 1), lambda b: (0, 0)),
        ],
        out_specs=[
            pl.BlockSpec((1, N, H), lambda b: (b, 0, 0)),
            pl.BlockSpec((1, N, 1), lambda b: (b, 0, 0)),
        ],
        out_shape=[
            jax.ShapeDtypeStruct((B, N, H), x.dtype),
            jax.ShapeDtypeStruct((B, N, 1), x.dtype),
        ],
        compiler_params=pltpu.CompilerParams(
            dimension_semantics=("parallel",)),
    )(x, occ, roots, w, u)
    return out, g


# speedup vs baseline: 1.5259x; 1.5259x over previous
import jax
import jax.numpy as jnp
from jax.experimental import pallas as pl
from jax.experimental.pallas import tpu as pltpu


def _occ_body(m_ref, occ_ref):
    bc = m_ref[0]
    rc = m_ref[1]
    ids = bc * 16 + rc
    widx = jax.lax.shift_right_logical(ids, 5)
    bit = jax.lax.shift_left(jnp.int32(1), ids & 31)

    words = []
    for k in range(8):
        sel = jnp.where(widx == k, bit, 0)
        while sel.shape[0] > 1:
            h = sel.shape[0] // 2
            sel = sel[:h] | sel[h:]
        words.append(sel)
    w8 = jnp.concatenate(words, axis=0)

    wrep = jnp.broadcast_to(w8[:, None, :], (8, 32, 128)).reshape(256, 128)
    t = jax.lax.broadcasted_iota(jnp.int32, (256, 128), 0) & 31
    bits = jax.lax.shift_right_logical(wrep, t) & 1
    occ_ref[...] = jnp.max(bits, axis=1, keepdims=True).astype(jnp.float32)


def _dense_body(x_ref, occ_ref, w_ref, u_ref, out_ref, g_ref):
    b = pl.program_id(0)
    H = w_ref.shape[0]
    wb = jnp.broadcast_to(w_ref[...], (H, H))

    x_blk = x_ref[0, 16:, :]
    gpre = jnp.dot(x_blk, wb, preferred_element_type=jnp.float32)
    g = jax.nn.sigmoid(gpre)
    out_ref[0, 16:, :] = x_blk * g
    g_ref[0, 16:, :] = g[:, 0:1]

    m16 = occ_ref[pl.ds(b * 16, 16), :]
    rv = x_ref[0, 0:1, :]
    a_b = jnp.sum(jnp.dot(rv, u_ref[...], preferred_element_type=jnp.float32))
    x16 = x_ref[0, 0:16, :]
    gpre16 = jnp.dot(x16, wb, preferred_element_type=jnp.float32)
    g16 = jax.nn.sigmoid(gpre16 + m16 * a_b)
    out_ref[0, 0:16, :] = x16 * g16 + (m16 * (1.0 - g16)) * rv
    g_ref[0, 0:16, :] = g16[:, 0:1]


def kernel(x, mask_nonzero, w, u):
    B, N, H = x.shape
    L = mask_nonzero.shape[1]
    m3 = mask_nonzero.reshape(2, L // 128, 128)

    occ = pl.pallas_call(
        _occ_body,
        out_shape=jax.ShapeDtypeStruct((256, 1), jnp.float32),
    )(m3)

    out, g = pl.pallas_call(
        _dense_body,
        grid=(B,),
        in_specs=[
            pl.BlockSpec((1, N, H), lambda b: (b, 0, 0)),
            pl.BlockSpec((256, 1), lambda b: (0, 0)),
            pl.BlockSpec((H, 1), lambda b: (0, 0)),
            pl.BlockSpec((H, 1), lambda b: (0, 0)),
        ],
        out_specs=[
            pl.BlockSpec((1, N, H), lambda b: (b, 0, 0)),
            pl.BlockSpec((1, N, 1), lambda b: (b, 0, 0)),
        ],
        out_shape=[
            jax.ShapeDtypeStruct((B, N, H), x.dtype),
            jax.ShapeDtypeStruct((B, N, 1), x.dtype),
        ],
        compiler_params=pltpu.CompilerParams(
            dimension_semantics=("parallel",)),
    )(x, occ, w, u)
    return out, g


# speedup vs baseline: 1.6372x; 1.0729x over previous
import jax
import jax.numpy as jnp
from jax.experimental import pallas as pl
from jax.experimental.pallas import tpu as pltpu

_BB = 4


def _occ_body(m_ref, occ_ref):
    bc = m_ref[0]
    rc = m_ref[1]
    ids = bc * 16 + rc
    widx = jax.lax.shift_right_logical(ids, 5)
    bit = jax.lax.shift_left(jnp.int32(1), ids & 31)

    words = []
    for k in range(8):
        sel = jnp.where(widx == k, bit, 0)
        while sel.shape[0] > 1:
            h = sel.shape[0] // 2
            sel = sel[:h] | sel[h:]
        words.append(sel)
    w8 = jnp.concatenate(words, axis=0)

    wrep = jnp.broadcast_to(w8[:, None, :], (8, 32, 128)).reshape(256, 128)
    t = jax.lax.broadcasted_iota(jnp.int32, (256, 128), 0) & 31
    bits = jax.lax.shift_right_logical(wrep, t) & 1
    occ_ref[...] = jnp.max(bits, axis=1, keepdims=True).astype(jnp.float32)


def _dense_body(x_ref, occ_ref, w_ref, u_ref, out_ref, g_ref):
    b = pl.program_id(0)
    H = w_ref.shape[0]
    wb = jnp.broadcast_to(w_ref[...], (H, H))

    for bb in range(_BB):
        x_blk = x_ref[bb, 16:, :]
        gpre = jnp.dot(x_blk, wb, preferred_element_type=jnp.float32)
        g = jax.nn.sigmoid(gpre)
        out_ref[bb, 16:, :] = x_blk * g
        g_ref[bb, 16:, :] = g[:, 0:1]

        m16 = occ_ref[pl.ds((b * _BB + bb) * 16, 16), :]
        rv = x_ref[bb, 0:1, :]
        a_b = jnp.sum(jnp.dot(rv, u_ref[...],
                              preferred_element_type=jnp.float32))
        x16 = x_ref[bb, 0:16, :]
        gpre16 = jnp.dot(x16, wb, preferred_element_type=jnp.float32)
        g16 = jax.nn.sigmoid(gpre16 + m16 * a_b)
        out_ref[bb, 0:16, :] = x16 * g16 + (m16 * (1.0 - g16)) * rv
        g_ref[bb, 0:16, :] = g16[:, 0:1]


def kernel(x, mask_nonzero, w, u):
    B, N, H = x.shape
    L = mask_nonzero.shape[1]
    m3 = mask_nonzero.reshape(2, L // 128, 128)

    occ = pl.pallas_call(
        _occ_body,
        out_shape=jax.ShapeDtypeStruct((256, 1), jnp.float32),
    )(m3)

    out, g = pl.pallas_call(
        _dense_body,
        grid=(B // _BB,),
        in_specs=[
            pl.BlockSpec((_BB, N, H), lambda b: (b, 0, 0)),
            pl.BlockSpec((256, 1), lambda b: (0, 0)),
            pl.BlockSpec((H, 1), lambda b: (0, 0)),
            pl.BlockSpec((H, 1), lambda b: (0, 0)),
        ],
        out_specs=[
            pl.BlockSpec((_BB, N, H), lambda b: (b, 0, 0)),
            pl.BlockSpec((_BB, N, 1), lambda b: (b, 0, 0)),
        ],
        out_shape=[
            jax.ShapeDtypeStruct((B, N, H), x.dtype),
            jax.ShapeDtypeStruct((B, N, 1), x.dtype),
        ],
        compiler_params=pltpu.CompilerParams(
            dimension_semantics=("parallel",)),
    )(x, occ, w, u)
    return out, g
